# Initial kernel scaffold; baseline (speedup 1.0000x reference)
import jax
import jax.numpy as jnp
from jax.experimental import pallas as pl


def kernel(x, edge_index, W1, b1, W2, b2):
    raise NotImplementedError("write your pallas kernel here")


# speedup vs baseline: 13.1324x; 13.1324x over previous
import functools

import jax
import jax.numpy as jnp
from jax import lax
from jax.experimental import pallas as pl
from jax.experimental.pallas import tpu as pltpu
from jax.experimental.pallas import tpu_sc as plsc

N = 10000
E = 320000
D = 128

NC = 2
NS = 16
NW = NC * NS
EPW = E // NW
CHUNK = 80
NCHUNK = EPW // CHUNK
NPAD = 10240
RPT = NPAD // NS
ZROWS = 128
DEGW = 16

_mesh = plsc.VectorSubcoreMesh(core_axis_name="c", subcore_axis_name="s")


@functools.partial(
    pl.kernel,
    out_type=jax.ShapeDtypeStruct((NC * NPAD, DEGW), jnp.float32),
    mesh=_mesh,
    scratch_types=[
        pltpu.VMEM((CHUNK,), jnp.int32),
        pltpu.VMEM((CHUNK, DEGW), jnp.float32),
        pltpu.VMEM((ZROWS, DEGW), jnp.float32),
        pltpu.VMEM_SHARED((NPAD, DEGW), jnp.float32),
    ],
)
def _sc_degree(dst_hbm, out_hbm, dst_v, ones_v, zbuf, acc_sh):
    c = lax.axis_index("c")
    s = lax.axis_index("s")
    wid = s * NC + c

    def fill_ones(r, carry):
        ones_v[r] = jnp.ones((DEGW,), jnp.float32)
        return carry

    lax.fori_loop(0, CHUNK, fill_ones, 0)

    def fill_zeros(r, carry):
        zbuf[r] = jnp.zeros((DEGW,), jnp.float32)
        return carry

    lax.fori_loop(0, ZROWS, fill_zeros, 0)

    base_row = s * RPT
    for j in range(RPT // ZROWS):
        pltpu.sync_copy(zbuf, acc_sh.at[pl.ds(base_row + j * ZROWS, ZROWS)])
    plsc.subcore_barrier()

    def body(i, carry):
        off = wid * EPW + i * CHUNK
        pltpu.sync_copy(dst_hbm.at[pl.ds(off, CHUNK)], dst_v)
        pltpu.sync_copy(ones_v, acc_sh.at[dst_v], add=True)
        return carry

    lax.fori_loop(0, NCHUNK, body, 0)
    plsc.subcore_barrier()
    pltpu.sync_copy(acc_sh.at[pl.ds(base_row, RPT)],
                    out_hbm.at[pl.ds(c * NPAD + base_row, RPT)])


@functools.partial(
    pl.kernel,
    out_type=jax.ShapeDtypeStruct((NC * NPAD, D), jnp.float32),
    mesh=_mesh,
    scratch_types=[
        pltpu.VMEM((CHUNK,), jnp.int32),
        pltpu.VMEM((CHUNK,), jnp.int32),
        pltpu.VMEM((CHUNK, D), jnp.float32),
        pltpu.VMEM((ZROWS, D), jnp.float32),
        pltpu.VMEM_SHARED((NPAD, D), jnp.float32),
        pltpu.SemaphoreType.DMA,
    ],
)
def _sc_agg(y_hbm, src_hbm, dst_hbm, out_hbm, src_v, dst_v, rows_v, zbuf,
            acc_sh, sem):
    c = lax.axis_index("c")
    s = lax.axis_index("s")
    wid = s * NC + c

    def fill_zeros(r, carry):
        for k in range(D // 16):
            zbuf[r, pl.ds(k * 16, 16)] = jnp.zeros((16,), jnp.float32)
        return carry

    lax.fori_loop(0, ZROWS, fill_zeros, 0)

    base_row = s * RPT
    for j in range(RPT // ZROWS):
        pltpu.sync_copy(zbuf, acc_sh.at[pl.ds(base_row + j * ZROWS, ZROWS)])
    plsc.subcore_barrier()

    def body(i, carry):
        off = wid * EPW + i * CHUNK
        pltpu.sync_copy(src_hbm.at[pl.ds(off, CHUNK)], src_v)
        pltpu.sync_copy(dst_hbm.at[pl.ds(off, CHUNK)], dst_v)
        pltpu.async_copy(y_hbm.at[src_v], rows_v, sem).wait()
        pltpu.sync_copy(rows_v, acc_sh.at[dst_v], add=True)
        return carry

    lax.fori_loop(0, NCHUNK, body, 0)
    plsc.subcore_barrier()
    pltpu.sync_copy(acc_sh.at[pl.ds(base_row, RPT)],
                    out_hbm.at[pl.ds(c * NPAD + base_row, RPT)])


_R = 1000


def _tc1_body(degp_ref, x_ref, w_ref, y_ref, dinv_ref):
    deg = degp_ref[0, :, :1] + degp_ref[1, :, :1] + 1.0
    dinv = lax.rsqrt(deg)
    y_ref[...] = jnp.dot(x_ref[...], w_ref[...],
                         preferred_element_type=jnp.float32) * dinv
    dinv_ref[...] = jnp.broadcast_to(dinv, (_R, DEGW))


def _tc1(degp, x, W1):
    return pl.pallas_call(
        _tc1_body,
        grid=(N // _R,),
        in_specs=[
            pl.BlockSpec((NC, _R, DEGW), lambda i: (0, i, 0)),
            pl.BlockSpec((_R, D), lambda i: (i, 0)),
            pl.BlockSpec((D, D), lambda i: (0, 0)),
        ],
        out_specs=[
            pl.BlockSpec((_R, D), lambda i: (i, 0)),
            pl.BlockSpec((_R, DEGW), lambda i: (i, 0)),
        ],
        out_shape=[
            jax.ShapeDtypeStruct((N, D), jnp.float32),
            jax.ShapeDtypeStruct((N, DEGW), jnp.float32),
        ],
    )(degp, x, W1)


def _tc2_body(p_ref, y1_ref, dinv_ref, b1_ref, w2_ref, y2_ref):
    dinv = dinv_ref[...][:, :1]
    agg = p_ref[0] + p_ref[1] + y1_ref[...]
    z = jnp.maximum(agg * dinv + b1_ref[...], 0.0)
    y2_ref[...] = jnp.dot(z, w2_ref[...],
                          preferred_element_type=jnp.float32) * dinv


def _tc2(p1, y1, dinv16, b1, W2):
    return pl.pallas_call(
        _tc2_body,
        grid=(N // _R,),
        in_specs=[
            pl.BlockSpec((NC, _R, D), lambda i: (0, i, 0)),
            pl.BlockSpec((_R, D), lambda i: (i, 0)),
            pl.BlockSpec((_R, DEGW), lambda i: (i, 0)),
            pl.BlockSpec((1, D), lambda i: (0, 0)),
            pl.BlockSpec((D, D), lambda i: (0, 0)),
        ],
        out_specs=pl.BlockSpec((_R, D), lambda i: (i, 0)),
        out_shape=jax.ShapeDtypeStruct((N, D), jnp.float32),
    )(p1, y1, dinv16, b1, W2)


def _tc3_body(p_ref, y2_ref, dinv_ref, b2_ref, out_ref):
    dinv = dinv_ref[...][:, :1]
    out_ref[...] = (p_ref[0] + p_ref[1] + y2_ref[...]) * dinv + b2_ref[...]


def _tc3(p2, y2, dinv16, b2):
    return pl.pallas_call(
        _tc3_body,
        grid=(N // _R,),
        in_specs=[
            pl.BlockSpec((NC, _R, D), lambda i: (0, i, 0)),
            pl.BlockSpec((_R, D), lambda i: (i, 0)),
            pl.BlockSpec((_R, DEGW), lambda i: (i, 0)),
            pl.BlockSpec((1, D), lambda i: (0, 0)),
        ],
        out_specs=pl.BlockSpec((_R, D), lambda i: (i, 0)),
        out_shape=jax.ShapeDtypeStruct((N, D), jnp.float32),
    )(p2, y2, dinv16, b2)


def kernel(x, edge_index, W1, b1, W2, b2):
    src = edge_index[0]
    dst = edge_index[1]
    degp = _sc_degree(dst).reshape(NC, NPAD, DEGW)
    y1, dinv16 = _tc1(degp, x, W1)
    p1 = _sc_agg(y1, src, dst).reshape(NC, NPAD, D)
    y2 = _tc2(p1, y1, dinv16, b1.reshape(1, D), W2)
    p2 = _sc_agg(y2, src, dst).reshape(NC, NPAD, D)
    return _tc3(p2, y2, dinv16, b2.reshape(1, D))


# speedup vs baseline: 27.7296x; 2.1115x over previous
import functools

import jax
import jax.numpy as jnp
from jax import lax
from jax.experimental import pallas as pl
from jax.experimental.pallas import tpu as pltpu
from jax.experimental.pallas import tpu_sc as plsc

N = 10000
E = 320000
D = 128

NC = 2
NS = 16
NW = NC * NS
EPW = E // NW
CHUNK = 80
NCHUNK = EPW // CHUNK
NPIPE = NCHUNK - 1
NBUF = 4
NPAD = 10240
RPT = NPAD // NS
ZROWS = 32
DEGW = 16

_mesh = plsc.VectorSubcoreMesh(core_axis_name="c", subcore_axis_name="s")


@functools.partial(
    pl.kernel,
    out_type=jax.ShapeDtypeStruct((NC * NPAD, DEGW), jnp.float32),
    mesh=_mesh,
    scratch_types=[
        pltpu.VMEM((NCHUNK, CHUNK), jnp.int32),
        pltpu.VMEM((CHUNK, DEGW), jnp.float32),
        pltpu.VMEM((ZROWS, DEGW), jnp.float32),
        pltpu.VMEM_SHARED((NPAD, DEGW), jnp.float32),
        [pltpu.SemaphoreType.DMA] * NBUF,
    ],
)
def _sc_degree(dst_hbm, out_hbm, dst_v, ones_v, zbuf, acc_sh, sems):
    c = lax.axis_index("c")
    s = lax.axis_index("s")
    wid = s * NC + c

    def fill_ones(r, carry):
        ones_v[r] = jnp.ones((DEGW,), jnp.float32)
        return carry

    lax.fori_loop(0, CHUNK, fill_ones, 0)

    def fill_zeros(r, carry):
        zbuf[r] = jnp.zeros((DEGW,), jnp.float32)
        return carry

    lax.fori_loop(0, ZROWS, fill_zeros, 0)

    pltpu.sync_copy(dst_hbm.at[wid], dst_v)
    base_row = s * RPT
    for j in range(RPT // ZROWS):
        pltpu.sync_copy(zbuf, acc_sh.at[pl.ds(base_row + j * ZROWS, ZROWS)])
    plsc.subcore_barrier()

    def body(g, carry):
        base = g * NBUF
        for b in range(NBUF):
            cc = base + b

            @pl.when(g > 0)
            def _wait():
                pltpu.make_async_copy(
                    ones_v, acc_sh.at[dst_v.at[cc - NBUF]], sems[b]).wait()

            pltpu.async_copy(ones_v, acc_sh.at[dst_v.at[cc]], sems[b],
                             add=True)
        return carry

    lax.fori_loop(0, NCHUNK // NBUF, body, 0)
    for b in range(NBUF):
        cc = (NCHUNK // NBUF - 1) * NBUF + b
        pltpu.make_async_copy(ones_v, acc_sh.at[dst_v.at[cc]], sems[b]).wait()
    pltpu.sync_copy(ones_v, acc_sh.at[dst_v.at[NCHUNK - 1]], add=True)
    plsc.subcore_barrier()
    pltpu.sync_copy(acc_sh.at[pl.ds(base_row, RPT)],
                    out_hbm.at[pl.ds(c * NPAD + base_row, RPT)])


@functools.partial(
    pl.kernel,
    out_type=jax.ShapeDtypeStruct((NC * NPAD, D), jnp.float32),
    mesh=_mesh,
    scratch_types=[
        pltpu.VMEM((NBUF, CHUNK), jnp.int32),
        pltpu.VMEM((NBUF, CHUNK), jnp.int32),
        pltpu.VMEM((NBUF, CHUNK, D), jnp.float32),
        pltpu.VMEM((ZROWS, D), jnp.float32),
        pltpu.VMEM_SHARED((NPAD, D), jnp.float32),
        [pltpu.SemaphoreType.DMA] * NBUF,
        [pltpu.SemaphoreType.DMA] * NBUF,
        [pltpu.SemaphoreType.DMA] * NBUF,
    ],
)
def _sc_agg(y_hbm, src_hbm, dst_hbm, out_hbm, src_v, dst_v, rows_v, zbuf,
            acc_sh, isems, gsems, ssems):
    c = lax.axis_index("c")
    s = lax.axis_index("s")
    wid = s * NC + c
    ebase = wid * EPW

    def fill_zeros(r, carry):
        for k in range(D // 16):
            zbuf[r, pl.ds(k * 16, 16)] = jnp.zeros((16,), jnp.float32)
        return carry

    lax.fori_loop(0, ZROWS, fill_zeros, 0)

    base_row = s * RPT
    for j in range(RPT // ZROWS):
        pltpu.sync_copy(zbuf, acc_sh.at[pl.ds(base_row + j * ZROWS, ZROWS)])

    def i_issue(cc, b):
        off = ebase + cc * CHUNK
        pltpu.async_copy(src_hbm.at[pl.ds(off, CHUNK)], src_v.at[b], isems[b])
        pltpu.async_copy(dst_hbm.at[pl.ds(off, CHUNK)], dst_v.at[b], isems[b])

    def i_wait(cc, b):
        off = ebase + cc * CHUNK
        pltpu.make_async_copy(src_hbm.at[pl.ds(off, CHUNK)], src_v.at[b],
                              isems[b]).wait()
        pltpu.make_async_copy(dst_hbm.at[pl.ds(off, CHUNK)], dst_v.at[b],
                              isems[b]).wait()

    def g_issue(cc, b):
        pltpu.async_copy(y_hbm.at[src_v.at[b]], rows_v.at[b], gsems[b])

    def g_wait(cc, b):
        pltpu.make_async_copy(y_hbm.at[src_v.at[b]], rows_v.at[b],
                              gsems[b]).wait()

    def s_issue(cc, b):
        pltpu.async_copy(rows_v.at[b], acc_sh.at[dst_v.at[b]], ssems[b],
                         add=True)

    def s_wait(cc, b):
        pltpu.make_async_copy(rows_v.at[b], acc_sh.at[dst_v.at[b]],
                              ssems[b]).wait()

    for b in range(NBUF):
        i_issue(b, b)
    plsc.subcore_barrier()

    def body(g, carry):
        base = g * NBUF
        for b in range(NBUF):
            cc = base + b
            i_wait(cc, b)
            g_issue(cc, b)
        for b in range(NBUF):
            cc = base + b
            g_wait(cc, b)
            s_issue(cc, b)
        for b in range(NBUF):
            cc = base + b
            s_wait(cc, b)

            @pl.when(cc + NBUF < NPIPE)
            def _issue():
                i_issue(cc + NBUF, b)
        return carry

    lax.fori_loop(0, NPIPE // NBUF, body, 0)
    i_issue(NCHUNK - 1, 0)
    i_wait(NCHUNK - 1, 0)
    pltpu.async_copy(y_hbm.at[src_v.at[0]], rows_v.at[0], gsems[0]).wait()
    pltpu.async_copy(rows_v.at[0], acc_sh.at[dst_v.at[0]], ssems[0],
                     add=True).wait()
    plsc.subcore_barrier()
    pltpu.sync_copy(acc_sh.at[pl.ds(base_row, RPT)],
                    out_hbm.at[pl.ds(c * NPAD + base_row, RPT)])


_R = 1000


def _tc1_body(degp_ref, x_ref, w_ref, y_ref, dinv_ref):
    deg = degp_ref[0, :, :1] + degp_ref[1, :, :1] + 1.0
    dinv = lax.rsqrt(deg)
    y_ref[...] = jnp.dot(x_ref[...], w_ref[...],
                         preferred_element_type=jnp.float32) * dinv
    dinv_ref[...] = jnp.broadcast_to(dinv, (_R, DEGW))


def _tc1(degp, x, W1):
    return pl.pallas_call(
        _tc1_body,
        grid=(N // _R,),
        in_specs=[
            pl.BlockSpec((NC, _R, DEGW), lambda i: (0, i, 0)),
            pl.BlockSpec((_R, D), lambda i: (i, 0)),
            pl.BlockSpec((D, D), lambda i: (0, 0)),
        ],
        out_specs=[
            pl.BlockSpec((_R, D), lambda i: (i, 0)),
            pl.BlockSpec((_R, DEGW), lambda i: (i, 0)),
        ],
        out_shape=[
            jax.ShapeDtypeStruct((N, D), jnp.float32),
            jax.ShapeDtypeStruct((N, DEGW), jnp.float32),
        ],
    )(degp, x, W1)


def _tc2_body(p_ref, y1_ref, dinv_ref, b1_ref, w2_ref, y2_ref):
    dinv = dinv_ref[...][:, :1]
    agg = p_ref[0] + p_ref[1] + y1_ref[...]
    z = jnp.maximum(agg * dinv + b1_ref[...], 0.0)
    y2_ref[...] = jnp.dot(z, w2_ref[...],
                          preferred_element_type=jnp.float32) * dinv


def _tc2(p1, y1, dinv16, b1, W2):
    return pl.pallas_call(
        _tc2_body,
        grid=(N // _R,),
        in_specs=[
            pl.BlockSpec((NC, _R, D), lambda i: (0, i, 0)),
            pl.BlockSpec((_R, D), lambda i: (i, 0)),
            pl.BlockSpec((_R, DEGW), lambda i: (i, 0)),
            pl.BlockSpec((1, D), lambda i: (0, 0)),
            pl.BlockSpec((D, D), lambda i: (0, 0)),
        ],
        out_specs=pl.BlockSpec((_R, D), lambda i: (i, 0)),
        out_shape=jax.ShapeDtypeStruct((N, D), jnp.float32),
    )(p1, y1, dinv16, b1, W2)


def _tc3_body(p_ref, y2_ref, dinv_ref, b2_ref, out_ref):
    dinv = dinv_ref[...][:, :1]
    out_ref[...] = (p_ref[0] + p_ref[1] + y2_ref[...]) * dinv + b2_ref[...]


def _tc3(p2, y2, dinv16, b2):
    return pl.pallas_call(
        _tc3_body,
        grid=(N // _R,),
        in_specs=[
            pl.BlockSpec((NC, _R, D), lambda i: (0, i, 0)),
            pl.BlockSpec((_R, D), lambda i: (i, 0)),
            pl.BlockSpec((_R, DEGW), lambda i: (i, 0)),
            pl.BlockSpec((1, D), lambda i: (0, 0)),
        ],
        out_specs=pl.BlockSpec((_R, D), lambda i: (i, 0)),
        out_shape=jax.ShapeDtypeStruct((N, D), jnp.float32),
    )(p2, y2, dinv16, b2)


def kernel(x, edge_index, W1, b1, W2, b2):
    src = edge_index[0]
    dst = edge_index[1]
    degp = _sc_degree(dst.reshape(NW, NCHUNK, CHUNK)).reshape(NC, NPAD, DEGW)
    y1, dinv16 = _tc1(degp, x, W1)
    p1 = _sc_agg(y1, src, dst).reshape(NC, NPAD, D)
    y2 = _tc2(p1, y1, dinv16, b1.reshape(1, D), W2)
    p2 = _sc_agg(y2, src, dst).reshape(NC, NPAD, D)
    return _tc3(p2, y2, dinv16, b2.reshape(1, D))
